# baseline (speedup 1.0000x reference)
import jax
import jax.numpy as jnp
from jax.experimental import pallas as pl
from jax.experimental.pallas import tpu as pltpu

N = 10000
NFEAT = 128
NHID = 16
NCLASS = 8
NCAT = NHID + NCLASS
R = 128
B = (N + R - 1) // R
NPAD = B * R
R2 = 1024
BR2 = NPAD // R2
C = 2048
BC = NPAD // C


def _pass1(adj_ref, x_ref, w1_ref, b1_ref, w2_ref, b2_ref,
           s2_ref, part_ref, cat_s):
    i = pl.program_id(0)

    @pl.when(i == 0)
    def _():
        s1 = jnp.dot(x_ref[...], w1_ref[...],
                     preferred_element_type=jnp.float32)
        cat_s[...] = jnp.zeros((NPAD, NCAT), jnp.float32)
        cat_s[0:N, :] = jnp.concatenate(
            [s1, jnp.zeros((N, NCLASS), jnp.float32)], axis=1)

    ablk = adj_ref[...]
    comb = jnp.dot(ablk, cat_s[0:N, :], preferred_element_type=jnp.float32)
    h = jnp.maximum(comb[:, :NHID] + b1_ref[...], 0.0)
    s2b = jnp.dot(h, w2_ref[...], preferred_element_type=jnp.float32)
    row = jax.lax.broadcasted_iota(jnp.int32, s2b.shape, 0) + i * R
    s2b = jnp.where(row < N, s2b, 0.0)
    rows = cat_s[pl.ds(i * R, R), :]
    cat_s[pl.ds(i * R, R), :] = jnp.concatenate([rows[:, :NHID], s2b], axis=1)
    part_ref[...] = comb[:, NHID:] + b2_ref[...]
    s2_ref[...] = s2b


def _pass2(adj_ref, s2_ref, part_ref, out_ref):
    i = pl.program_id(0)
    j = pl.program_id(1)

    @pl.when(j == 0)
    def _():
        out_ref[...] = part_ref[...]

    def _accum(mask_adj_tail):
        ablk = adj_ref[...]
        if mask_adj_tail:
            col = jax.lax.broadcasted_iota(jnp.int32, (R2, C), 1) + j * C
            ablk = jnp.where(col < N, ablk, 0.0)
        s2b = s2_ref[pl.ds(j * C, C), :]
        srow = jax.lax.broadcasted_iota(jnp.int32, s2b.shape, 0) + j * C
        for g in range(R2 // R):
            s2g = jnp.where(srow >= i * R2 + g * R, s2b, 0.0)
            out_ref[g * R:(g + 1) * R, :] += jnp.dot(
                ablk[g * R:(g + 1) * R, :], s2g,
                preferred_element_type=jnp.float32)

    needed = j >= i * R2 // C

    @pl.when(needed & (j < BC - 1))
    def _():
        _accum(False)

    @pl.when(needed & (j == BC - 1))
    def _():
        _accum(True)


def _upper_or_first(i, j):
    return i, jnp.maximum(j, i * R2 // C)


def kernel(x, adj, W1, b1, W2, b2):
    b1r = b1.reshape(1, NHID)
    b2r = b2.reshape(1, NCLASS)

    s2, part = pl.pallas_call(
        _pass1,
        grid=(B,),
        in_specs=[
            pl.BlockSpec((R, N), lambda i: (i, 0)),
            pl.BlockSpec((N, NFEAT), lambda i: (0, 0)),
            pl.BlockSpec((NFEAT, NHID), lambda i: (0, 0)),
            pl.BlockSpec((1, NHID), lambda i: (0, 0)),
            pl.BlockSpec((NHID, NCLASS), lambda i: (0, 0)),
            pl.BlockSpec((1, NCLASS), lambda i: (0, 0)),
        ],
        out_specs=[
            pl.BlockSpec((R, NCLASS), lambda i: (i, 0)),
            pl.BlockSpec((R, NCLASS), lambda i: (i, 0)),
        ],
        out_shape=[
            jax.ShapeDtypeStruct((NPAD, NCLASS), jnp.float32),
            jax.ShapeDtypeStruct((NPAD, NCLASS), jnp.float32),
        ],
        scratch_shapes=[
            pltpu.VMEM((NPAD, NCAT), jnp.float32),
        ],
    )(adj, x, W1, b1r, W2, b2r)

    out = pl.pallas_call(
        _pass2,
        grid=(BR2, BC),
        in_specs=[
            pl.BlockSpec((R2, C), _upper_or_first),
            pl.BlockSpec((NPAD, NCLASS), lambda i, j: (0, 0)),
            pl.BlockSpec((R2, NCLASS), lambda i, j: (i, 0)),
        ],
        out_specs=pl.BlockSpec((R2, NCLASS), lambda i, j: (i, 0)),
        out_shape=jax.ShapeDtypeStruct((N, NCLASS), jnp.float32),
        compiler_params=pltpu.CompilerParams(
            dimension_semantics=("parallel", "arbitrary")),
    )(adj, s2, part)
    return out


# speedup vs baseline: 1.0174x; 1.0174x over previous
import jax
import jax.numpy as jnp
from jax.experimental import pallas as pl
from jax.experimental.pallas import tpu as pltpu

N = 10000
NFEAT = 128
NHID = 16
NCLASS = 8
NCAT = NHID + NCLASS
R = 256
B = (N + R - 1) // R
NPAD = B * R
R2 = 2048
BR2 = NPAD // R2
C = 2048
BC = NPAD // C
assert NPAD % R2 == 0 and NPAD % C == 0 and R2 % R == 0


def _pass1(adj_ref, x_ref, w1_ref, b1_ref, w2_ref, b2_ref,
           s2_ref, part_ref, cat_s):
    i = pl.program_id(0)

    @pl.when(i == 0)
    def _():
        s1 = jnp.dot(x_ref[...], w1_ref[...],
                     preferred_element_type=jnp.float32)
        cat_s[...] = jnp.zeros((NPAD, NCAT), jnp.float32)
        cat_s[0:N, :] = jnp.concatenate(
            [s1, jnp.zeros((N, NCLASS), jnp.float32)], axis=1)

    ablk = adj_ref[...]
    comb = jnp.dot(ablk, cat_s[0:N, :], preferred_element_type=jnp.float32)
    h = jnp.maximum(comb[:, :NHID] + b1_ref[...], 0.0)
    s2b = jnp.dot(h, w2_ref[...], preferred_element_type=jnp.float32)
    row = jax.lax.broadcasted_iota(jnp.int32, s2b.shape, 0) + i * R
    s2b = jnp.where(row < N, s2b, 0.0)
    rows = cat_s[pl.ds(i * R, R), :]
    cat_s[pl.ds(i * R, R), :] = jnp.concatenate([rows[:, :NHID], s2b], axis=1)
    part_ref[...] = comb[:, NHID:] + b2_ref[...]
    s2_ref[...] = s2b


def _pass2(adj_ref, s2_ref, part_ref, out_ref):
    i = pl.program_id(0)
    j = pl.program_id(1)

    @pl.when(j == 0)
    def _():
        out_ref[...] = part_ref[...]

    def _accum(mask_adj_tail):
        ablk = adj_ref[...]
        if mask_adj_tail:
            col = jax.lax.broadcasted_iota(jnp.int32, (R2, C), 1) + j * C
            ablk = jnp.where(col < N, ablk, 0.0)
        s2b = s2_ref[pl.ds(j * C, C), :]
        srow = jax.lax.broadcasted_iota(jnp.int32, s2b.shape, 0) + j * C
        for g in range(R2 // R):
            s2g = jnp.where(srow >= i * R2 + g * R, s2b, 0.0)
            out_ref[g * R:(g + 1) * R, :] += jnp.dot(
                ablk[g * R:(g + 1) * R, :], s2g,
                preferred_element_type=jnp.float32)

    needed = j >= i * R2 // C

    @pl.when(needed & (j < BC - 1))
    def _():
        _accum(False)

    @pl.when(needed & (j == BC - 1))
    def _():
        _accum(True)


def _upper_or_first(i, j):
    return i, jnp.maximum(j, i * R2 // C)


def kernel(x, adj, W1, b1, W2, b2):
    b1r = b1.reshape(1, NHID)
    b2r = b2.reshape(1, NCLASS)

    s2, part = pl.pallas_call(
        _pass1,
        grid=(B,),
        in_specs=[
            pl.BlockSpec((R, N), lambda i: (i, 0)),
            pl.BlockSpec((N, NFEAT), lambda i: (0, 0)),
            pl.BlockSpec((NFEAT, NHID), lambda i: (0, 0)),
            pl.BlockSpec((1, NHID), lambda i: (0, 0)),
            pl.BlockSpec((NHID, NCLASS), lambda i: (0, 0)),
            pl.BlockSpec((1, NCLASS), lambda i: (0, 0)),
        ],
        out_specs=[
            pl.BlockSpec((R, NCLASS), lambda i: (i, 0)),
            pl.BlockSpec((R, NCLASS), lambda i: (i, 0)),
        ],
        out_shape=[
            jax.ShapeDtypeStruct((NPAD, NCLASS), jnp.float32),
            jax.ShapeDtypeStruct((NPAD, NCLASS), jnp.float32),
        ],
        scratch_shapes=[
            pltpu.VMEM((NPAD, NCAT), jnp.float32),
        ],
    )(adj, x, W1, b1r, W2, b2r)

    out = pl.pallas_call(
        _pass2,
        grid=(BR2, BC),
        in_specs=[
            pl.BlockSpec((R2, C), _upper_or_first),
            pl.BlockSpec((NPAD, NCLASS), lambda i, j: (0, 0)),
            pl.BlockSpec((R2, NCLASS), lambda i, j: (i, 0)),
        ],
        out_specs=pl.BlockSpec((R2, NCLASS), lambda i, j: (i, 0)),
        out_shape=jax.ShapeDtypeStruct((N, NCLASS), jnp.float32),
        compiler_params=pltpu.CompilerParams(
            dimension_semantics=("parallel", "arbitrary")),
    )(adj, s2, part)
    return out


# speedup vs baseline: 1.0180x; 1.0006x over previous
import jax
import jax.numpy as jnp
from jax.experimental import pallas as pl
from jax.experimental.pallas import tpu as pltpu

N = 10000
NFEAT = 128
NHID = 16
NCLASS = 8
NCAT = NHID + NCLASS
R = 512
B = (N + R - 1) // R
NPAD = B * R
R2 = 2048
BR2 = NPAD // R2
C = 2048
BC = NPAD // C
assert NPAD % R2 == 0 and NPAD % C == 0 and R2 % R == 0


def _pass1(adj_ref, x_ref, w1_ref, b1_ref, w2_ref, b2_ref,
           s2_ref, part_ref, cat_s):
    i = pl.program_id(0)

    @pl.when(i == 0)
    def _():
        s1 = jnp.dot(x_ref[...], w1_ref[...],
                     preferred_element_type=jnp.float32)
        cat_s[...] = jnp.zeros((NPAD, NCAT), jnp.float32)
        cat_s[0:N, :] = jnp.concatenate(
            [s1, jnp.zeros((N, NCLASS), jnp.float32)], axis=1)

    ablk = adj_ref[...]
    comb = jnp.dot(ablk, cat_s[0:N, :], preferred_element_type=jnp.float32)
    h = jnp.maximum(comb[:, :NHID] + b1_ref[...], 0.0)
    s2b = jnp.dot(h, w2_ref[...], preferred_element_type=jnp.float32)
    row = jax.lax.broadcasted_iota(jnp.int32, s2b.shape, 0) + i * R
    s2b = jnp.where(row < N, s2b, 0.0)
    rows = cat_s[pl.ds(i * R, R), :]
    cat_s[pl.ds(i * R, R), :] = jnp.concatenate([rows[:, :NHID], s2b], axis=1)
    part_ref[...] = comb[:, NHID:] + b2_ref[...]
    s2_ref[...] = s2b


def _pass2(adj_ref, s2_ref, part_ref, out_ref):
    i = pl.program_id(0)
    j = pl.program_id(1)

    @pl.when(j == 0)
    def _():
        out_ref[...] = part_ref[...]

    def _accum(mask_adj_tail):
        ablk = adj_ref[...]
        if mask_adj_tail:
            col = jax.lax.broadcasted_iota(jnp.int32, (R2, C), 1) + j * C
            ablk = jnp.where(col < N, ablk, 0.0)
        s2b = s2_ref[pl.ds(j * C, C), :]
        srow = jax.lax.broadcasted_iota(jnp.int32, s2b.shape, 0) + j * C
        for g in range(R2 // R):
            s2g = jnp.where(srow >= i * R2 + g * R, s2b, 0.0)
            out_ref[g * R:(g + 1) * R, :] += jnp.dot(
                ablk[g * R:(g + 1) * R, :], s2g,
                preferred_element_type=jnp.float32)

    needed = j >= i * R2 // C

    @pl.when(needed & (j < BC - 1))
    def _():
        _accum(False)

    @pl.when(needed & (j == BC - 1))
    def _():
        _accum(True)


def _upper_or_first(i, j):
    return i, jnp.maximum(j, i * R2 // C)


def kernel(x, adj, W1, b1, W2, b2):
    b1r = b1.reshape(1, NHID)
    b2r = b2.reshape(1, NCLASS)

    s2, part = pl.pallas_call(
        _pass1,
        grid=(B,),
        in_specs=[
            pl.BlockSpec((R, N), lambda i: (i, 0)),
            pl.BlockSpec((N, NFEAT), lambda i: (0, 0)),
            pl.BlockSpec((NFEAT, NHID), lambda i: (0, 0)),
            pl.BlockSpec((1, NHID), lambda i: (0, 0)),
            pl.BlockSpec((NHID, NCLASS), lambda i: (0, 0)),
            pl.BlockSpec((1, NCLASS), lambda i: (0, 0)),
        ],
        out_specs=[
            pl.BlockSpec((R, NCLASS), lambda i: (i, 0)),
            pl.BlockSpec((R, NCLASS), lambda i: (i, 0)),
        ],
        out_shape=[
            jax.ShapeDtypeStruct((NPAD, NCLASS), jnp.float32),
            jax.ShapeDtypeStruct((NPAD, NCLASS), jnp.float32),
        ],
        scratch_shapes=[
            pltpu.VMEM((NPAD, NCAT), jnp.float32),
        ],
    )(adj, x, W1, b1r, W2, b2r)

    out = pl.pallas_call(
        _pass2,
        grid=(BR2, BC),
        in_specs=[
            pl.BlockSpec((R2, C), _upper_or_first),
            pl.BlockSpec((NPAD, NCLASS), lambda i, j: (0, 0)),
            pl.BlockSpec((R2, NCLASS), lambda i, j: (i, 0)),
        ],
        out_specs=pl.BlockSpec((R2, NCLASS), lambda i, j: (i, 0)),
        out_shape=jax.ShapeDtypeStruct((N, NCLASS), jnp.float32),
        compiler_params=pltpu.CompilerParams(
            dimension_semantics=("parallel", "arbitrary")),
    )(adj, s2, part)
    return out
